# baseline (device time: 25431 ns/iter reference)
import jax
import jax.numpy as jnp
from jax import lax
from jax.experimental import pallas as pl
from jax.experimental.pallas import tpu as pltpu

N_DEV = 4
N_SUB = 2
N_XFER = 10


def kernel(x):
    m_per, n = x.shape
    m_c = m_per // N_DEV
    m_s = m_c // N_SUB
    n_h = n // 2

    def body(x_ref, out_ref, sbuf, rbuf, send_sems, recv_sems):
        my = lax.axis_index("i")
        left = lax.rem(my + N_DEV - 1, N_DEV)
        right = lax.rem(my + 1, N_DEV)

        A = pl.ds(0, n_h)
        B = pl.ds(n_h, n_h)

        def rows(c, k):
            return pl.ds(c * m_c + k * m_s, m_s)

        def sub(k):
            return pl.ds(k * m_s, m_s)

        c_m2 = lax.rem(my + 2, N_DEV)
        c_p1 = right
        c_m1 = left

        barrier_sem = pltpu.get_barrier_semaphore()
        for nbr in (left, right):
            pl.semaphore_signal(
                barrier_sem, inc=1,
                device_id=(nbr,), device_id_type=pl.DeviceIdType.MESH,
            )
        pl.semaphore_wait(barrier_sem, 2)

        pending = []

        def start(t, k, src, dst, target):
            rdma = pltpu.make_async_remote_copy(
                src_ref=src, dst_ref=dst,
                send_sem=send_sems.at[t, k],
                recv_sem=recv_sems.at[t, k],
                device_id=(target,),
                device_id_type=pl.DeviceIdType.MESH,
            )
            rdma.start()
            pending.append(rdma)
            return rdma

        T = {}

        for k in range(N_SUB):
            T[0, k] = start(0, k, x_ref.at[rows(c_m2, k), A],
                            rbuf.at[0, sub(k), :], right)
            T[3, k] = start(3, k, x_ref.at[rows(c_m2, k), B],
                            rbuf.at[3, sub(k), :], left)
            T[1, k] = start(1, k, x_ref.at[rows(c_p1, k), B],
                            rbuf.at[1, sub(k), :], right)
            T[2, k] = start(2, k, x_ref.at[rows(c_m1, k), A],
                            rbuf.at[2, sub(k), :], left)

        for k in range(N_SUB):
            T[0, k].wait_recv()
            sbuf[0, sub(k), :] = x_ref[rows(c_p1, k), A] + rbuf[0, sub(k), :]
            T[4, k] = start(4, k, sbuf.at[0, sub(k), :],
                            rbuf.at[4, sub(k), :], right)
            T[3, k].wait_recv()
            sbuf[1, sub(k), :] = x_ref[rows(c_m1, k), B] + rbuf[3, sub(k), :]
            T[5, k] = start(5, k, sbuf.at[1, sub(k), :],
                            rbuf.at[5, sub(k), :], left)

        for k in range(N_SUB):
            T[2, k].wait_recv()
            T[4, k].wait_recv()
            out_ref[rows(my, k), A] = (x_ref[rows(my, k), A]
                                       + rbuf[2, sub(k), :]
                                       + rbuf[4, sub(k), :])
            T[1, k].wait_recv()
            T[5, k].wait_recv()
            out_ref[rows(my, k), B] = (x_ref[rows(my, k), B]
                                       + rbuf[1, sub(k), :]
                                       + rbuf[5, sub(k), :])
            own = out_ref.at[rows(my, k), :]
            T[6, k] = start(6, k, own, own, right)
            T[7, k] = start(7, k, own, own, left)

        for k in range(N_SUB):
            T[6, k].wait_recv()
            fw = out_ref.at[rows(c_m1, k), A]
            T[8, k] = start(8, k, fw, fw, right)
            T[7, k].wait_recv()
            fw = out_ref.at[rows(c_p1, k), B]
            T[9, k] = start(9, k, fw, fw, left)

        for k in range(N_SUB):
            T[8, k].wait_recv()
            T[9, k].wait_recv()

        for rdma in pending:
            rdma.wait_send()

    return pl.pallas_call(
        body,
        out_shape=jax.ShapeDtypeStruct((m_per, n), x.dtype),
        in_specs=[pl.BlockSpec(memory_space=pltpu.VMEM)],
        out_specs=pl.BlockSpec(memory_space=pltpu.VMEM),
        scratch_shapes=[
            pltpu.VMEM((2, m_c, n // 2), x.dtype),
            pltpu.VMEM((6, m_c, n // 2), x.dtype),
            pltpu.SemaphoreType.DMA((N_XFER, N_SUB)),
            pltpu.SemaphoreType.DMA((N_XFER, N_SUB)),
        ],
        compiler_params=pltpu.CompilerParams(collective_id=0),
    )(x)


# device time: 24188 ns/iter; 1.0514x vs baseline; 1.0514x over previous
import jax
import jax.numpy as jnp
from jax import lax
from jax.experimental import pallas as pl
from jax.experimental.pallas import tpu as pltpu

N_DEV = 4
N_SUB = 4
N_XFER = 10


def kernel(x):
    m_per, n = x.shape
    m_c = m_per // N_DEV
    m_s = m_c // N_SUB
    n_h = n // 2

    def body(x_ref, out_ref, sbuf, rbuf, send_sems, recv_sems):
        my = lax.axis_index("i")
        left = lax.rem(my + N_DEV - 1, N_DEV)
        right = lax.rem(my + 1, N_DEV)

        A = pl.ds(0, n_h)
        B = pl.ds(n_h, n_h)

        def rows(c, k):
            return pl.ds(c * m_c + k * m_s, m_s)

        def sub(k):
            return pl.ds(k * m_s, m_s)

        c_m2 = lax.rem(my + 2, N_DEV)
        c_p1 = right
        c_m1 = left

        barrier_sem = pltpu.get_barrier_semaphore()
        for nbr in (left, right):
            pl.semaphore_signal(
                barrier_sem, inc=1,
                device_id=(nbr,), device_id_type=pl.DeviceIdType.MESH,
            )
        pl.semaphore_wait(barrier_sem, 2)

        pending = []

        def start(t, k, src, dst, target):
            rdma = pltpu.make_async_remote_copy(
                src_ref=src, dst_ref=dst,
                send_sem=send_sems.at[t, k],
                recv_sem=recv_sems.at[t, k],
                device_id=(target,),
                device_id_type=pl.DeviceIdType.MESH,
            )
            rdma.start()
            pending.append(rdma)
            return rdma

        T = {}

        for k in range(N_SUB):
            T[0, k] = start(0, k, x_ref.at[rows(c_m2, k), A],
                            rbuf.at[0, sub(k), :], right)
            T[3, k] = start(3, k, x_ref.at[rows(c_m2, k), B],
                            rbuf.at[3, sub(k), :], left)
        for k in range(N_SUB):
            T[1, k] = start(1, k, x_ref.at[rows(c_p1, k), B],
                            rbuf.at[1, sub(k), :], right)
            T[2, k] = start(2, k, x_ref.at[rows(c_m1, k), A],
                            rbuf.at[2, sub(k), :], left)

        for k in range(N_SUB):
            T[0, k].wait_recv()
            sbuf[0, sub(k), :] = x_ref[rows(c_p1, k), A] + rbuf[0, sub(k), :]
            T[4, k] = start(4, k, sbuf.at[0, sub(k), :],
                            rbuf.at[4, sub(k), :], right)
            T[3, k].wait_recv()
            sbuf[1, sub(k), :] = x_ref[rows(c_m1, k), B] + rbuf[3, sub(k), :]
            T[5, k] = start(5, k, sbuf.at[1, sub(k), :],
                            rbuf.at[5, sub(k), :], left)

        for k in range(N_SUB):
            T[2, k].wait_recv()
            T[4, k].wait_recv()
            out_ref[rows(my, k), A] = (x_ref[rows(my, k), A]
                                       + rbuf[2, sub(k), :]
                                       + rbuf[4, sub(k), :])
            T[1, k].wait_recv()
            T[5, k].wait_recv()
            out_ref[rows(my, k), B] = (x_ref[rows(my, k), B]
                                       + rbuf[1, sub(k), :]
                                       + rbuf[5, sub(k), :])
            own = out_ref.at[rows(my, k), :]
            T[6, k] = start(6, k, own, own, right)
            T[7, k] = start(7, k, own, own, left)

        for k in range(N_SUB):
            T[6, k].wait_recv()
            fw = out_ref.at[rows(c_m1, k), A]
            T[8, k] = start(8, k, fw, fw, right)
            T[7, k].wait_recv()
            fw = out_ref.at[rows(c_p1, k), B]
            T[9, k] = start(9, k, fw, fw, left)

        for k in range(N_SUB):
            T[8, k].wait_recv()
            T[9, k].wait_recv()

        for rdma in pending:
            rdma.wait_send()

    return pl.pallas_call(
        body,
        out_shape=jax.ShapeDtypeStruct((m_per, n), x.dtype),
        in_specs=[pl.BlockSpec(memory_space=pltpu.VMEM)],
        out_specs=pl.BlockSpec(memory_space=pltpu.VMEM),
        scratch_shapes=[
            pltpu.VMEM((2, m_c, n // 2), x.dtype),
            pltpu.VMEM((6, m_c, n // 2), x.dtype),
            pltpu.SemaphoreType.DMA((N_XFER, N_SUB)),
            pltpu.SemaphoreType.DMA((N_XFER, N_SUB)),
        ],
        compiler_params=pltpu.CompilerParams(collective_id=0),
    )(x)


# device time: 24184 ns/iter; 1.0516x vs baseline; 1.0002x over previous
import jax
import jax.numpy as jnp
from jax import lax
from jax.experimental import pallas as pl
from jax.experimental.pallas import tpu as pltpu

N_DEV = 4
N_SUB = 4
N_XFER = 10


def kernel(x):
    m_per, n = x.shape
    m_c = m_per // N_DEV
    m_s = m_c // N_SUB
    n_h = n // 2

    def body(x_ref, out_ref, sbuf, rbuf, send_sems, recv_sems):
        my = lax.axis_index("i")
        left = lax.rem(my + N_DEV - 1, N_DEV)
        right = lax.rem(my + 1, N_DEV)

        A = pl.ds(0, n_h)
        B = pl.ds(n_h, n_h)

        def rows(c, k):
            return pl.ds(c * m_c + k * m_s, m_s)

        def sub(k):
            return pl.ds(k * m_s, m_s)

        c_m2 = lax.rem(my + 2, N_DEV)
        c_p1 = right
        c_m1 = left

        barrier_sem = pltpu.get_barrier_semaphore()
        for nbr in (left, right):
            pl.semaphore_signal(
                barrier_sem, inc=1,
                device_id=(nbr,), device_id_type=pl.DeviceIdType.MESH,
            )
        pl.semaphore_wait(barrier_sem, 2)

        pending = []

        def start(t, k, src, dst, target):
            rdma = pltpu.make_async_remote_copy(
                src_ref=src, dst_ref=dst,
                send_sem=send_sems.at[t, k],
                recv_sem=recv_sems.at[t, k],
                device_id=(target,),
                device_id_type=pl.DeviceIdType.MESH,
            )
            rdma.start()
            pending.append(rdma)
            return rdma

        T = {}

        for k in range(N_SUB):
            T[0, k] = start(0, k, x_ref.at[rows(c_m2, k), A],
                            rbuf.at[0, sub(k), :], right)
            T[3, k] = start(3, k, x_ref.at[rows(c_m2, k), B],
                            rbuf.at[3, sub(k), :], left)
        for k in range(N_SUB):
            T[1, k] = start(1, k, x_ref.at[rows(c_p1, k), B],
                            rbuf.at[1, sub(k), :], right)
            T[2, k] = start(2, k, x_ref.at[rows(c_m1, k), A],
                            rbuf.at[2, sub(k), :], left)

        for k in range(N_SUB):
            T[0, k].wait_recv()
            sbuf[0, sub(k), :] = x_ref[rows(c_p1, k), A] + rbuf[0, sub(k), :]
            T[4, k] = start(4, k, sbuf.at[0, sub(k), :],
                            rbuf.at[4, sub(k), :], right)
            T[3, k].wait_recv()
            sbuf[1, sub(k), :] = x_ref[rows(c_m1, k), B] + rbuf[3, sub(k), :]
            T[5, k] = start(5, k, sbuf.at[1, sub(k), :],
                            rbuf.at[5, sub(k), :], left)

        for k in range(N_SUB):
            T[2, k].wait_recv()
            out_ref[rows(my, k), A] = (x_ref[rows(my, k), A]
                                       + rbuf[2, sub(k), :])
            T[1, k].wait_recv()
            out_ref[rows(my, k), B] = (x_ref[rows(my, k), B]
                                       + rbuf[1, sub(k), :])

        for k in range(N_SUB):
            T[4, k].wait_recv()
            out_ref[rows(my, k), A] = (out_ref[rows(my, k), A]
                                       + rbuf[4, sub(k), :])
            T[5, k].wait_recv()
            out_ref[rows(my, k), B] = (out_ref[rows(my, k), B]
                                       + rbuf[5, sub(k), :])
            own = out_ref.at[rows(my, k), :]
            T[6, k] = start(6, k, own, own, right)
            T[7, k] = start(7, k, own, own, left)

        for k in range(N_SUB):
            T[6, k].wait_recv()
            fw = out_ref.at[rows(c_m1, k), A]
            T[8, k] = start(8, k, fw, fw, right)
            T[7, k].wait_recv()
            fw = out_ref.at[rows(c_p1, k), B]
            T[9, k] = start(9, k, fw, fw, left)

        for k in range(N_SUB):
            T[8, k].wait_recv()
            T[9, k].wait_recv()

        for rdma in pending:
            rdma.wait_send()

    return pl.pallas_call(
        body,
        out_shape=jax.ShapeDtypeStruct((m_per, n), x.dtype),
        in_specs=[pl.BlockSpec(memory_space=pltpu.VMEM)],
        out_specs=pl.BlockSpec(memory_space=pltpu.VMEM),
        scratch_shapes=[
            pltpu.VMEM((2, m_c, n // 2), x.dtype),
            pltpu.VMEM((6, m_c, n // 2), x.dtype),
            pltpu.SemaphoreType.DMA((N_XFER, N_SUB)),
            pltpu.SemaphoreType.DMA((N_XFER, N_SUB)),
        ],
        compiler_params=pltpu.CompilerParams(collective_id=0),
    )(x)
